# Initial kernel scaffold; baseline (speedup 1.0000x reference)
import jax
import jax.numpy as jnp
from jax.experimental import pallas as pl


def kernel(features, edge_index, edgenet_input, cheb_w0, cheb_w_rest, pae_w1, pae_b1, pae_g, pae_bt, pae_w2, pae_b2, cls_w1, cls_b1, cls_g, cls_bt, cls_w2, cls_b2):
    raise NotImplementedError("write your pallas kernel here")


# speedup vs baseline: 1.3636x; 1.3636x over previous
import functools

import jax
import jax.numpy as jnp
from jax import lax
from jax.experimental import pallas as pl
from jax.experimental.pallas import tpu as pltpu
from jax.experimental.pallas import tpu_sc as plsc

N = 10000
E = 320000
D_IN = 128
NHID = 16
NGL = 4
EDGENET_DIM = 32
PAE_HID = 32
EPS = 1e-5

NW = 32
C = 128
NCH = 79
E_PAD = NW * NCH * C
N_PAD = 10240
SLICE = N_PAD // 16

_SC_MESH = dict(core_axis_name="c", subcore_axis_name="s")


def _deg_body(src_hbm, ew_hbm, zero_hbm, out_hbm, src_v, ew_v, acc_sh):
    cid = lax.axis_index("c")
    sid = lax.axis_index("s")
    wid = sid * 2 + cid
    pltpu.sync_copy(src_hbm.at[wid], src_v)
    pltpu.sync_copy(ew_hbm.at[wid], ew_v)
    pltpu.sync_copy(zero_hbm.at[pl.ds(sid * SLICE, SLICE)],
                    acc_sh.at[pl.ds(sid * SLICE, SLICE)])
    plsc.subcore_barrier()

    def chunk(ch, carry):
        pltpu.sync_copy(ew_v.at[ch], acc_sh.at[src_v.at[ch]], add=True)
        return carry

    lax.fori_loop(0, NCH, chunk, 0)
    plsc.subcore_barrier()
    pltpu.sync_copy(acc_sh.at[pl.ds(sid * SLICE, SLICE)],
                    out_hbm.at[cid, pl.ds(sid * SLICE, SLICE)])


_deg_kernel = functools.partial(
    pl.kernel,
    mesh=plsc.VectorSubcoreMesh(**_SC_MESH),
    out_type=jax.ShapeDtypeStruct((2, N_PAD), jnp.float32),
    scratch_types=[
        pltpu.VMEM((NCH, C), jnp.int32),
        pltpu.VMEM((NCH, C), jnp.float32),
        pltpu.VMEM_SHARED((N_PAD,), jnp.float32),
    ],
)(_deg_body)


def _make_scatter_kernel(F):
    def body(v_hbm, src_hbm, dst_hbm, ew_hbm, zero_hbm, out_hbm,
             src_v, dst_v, ew_v, rows_v, acc_sh, sem):
        cid = lax.axis_index("c")
        sid = lax.axis_index("s")
        wid = sid * 2 + cid
        pltpu.sync_copy(src_hbm.at[wid], src_v)
        pltpu.sync_copy(dst_hbm.at[wid], dst_v)
        pltpu.sync_copy(ew_hbm.at[wid], ew_v)
        pltpu.sync_copy(zero_hbm.at[pl.ds(sid * SLICE, SLICE)],
                        acc_sh.at[pl.ds(sid * SLICE, SLICE)])
        plsc.subcore_barrier()

        gdn = lax.GatherDimensionNumbers(
            offset_dims=(), collapsed_slice_dims=(0,), start_index_map=(0,))

        def splat(vec, j):
            idx = jnp.full((16, 1), j, jnp.int32)
            return lax.gather(vec, idx, gdn, (1,),
                              mode=lax.GatherScatterMode.PROMISE_IN_BOUNDS)

        def chunk(ch, carry):
            pltpu.async_copy(v_hbm.at[src_v.at[ch]], rows_v, sem).wait()

            def group(g, c2):
                ew16 = ew_v[pl.ds(ch * C + g * 16, 16)]
                for j in range(16):
                    nb = splat(ew16, j)
                    e = g * 16 + j
                    for h in range(F // 16):
                        sl = pl.ds(h * 16, 16)
                        rows_v[e, sl] = rows_v[e, sl] * nb
                return c2

            lax.fori_loop(0, C // 16, group, 0)
            pltpu.sync_copy(rows_v, acc_sh.at[dst_v.at[ch]], add=True)
            return carry

        lax.fori_loop(0, NCH, chunk, 0)
        plsc.subcore_barrier()
        pltpu.sync_copy(acc_sh.at[pl.ds(sid * SLICE, SLICE)],
                        out_hbm.at[cid, pl.ds(sid * SLICE, SLICE)])

    return functools.partial(
        pl.kernel,
        mesh=plsc.VectorSubcoreMesh(**_SC_MESH),
        out_type=jax.ShapeDtypeStruct((2, N_PAD, F), jnp.float32),
        compiler_params=pltpu.CompilerParams(use_tc_tiling_on_sc=True),
        scratch_types=[
            pltpu.VMEM((NCH, C), jnp.int32),
            pltpu.VMEM((NCH, C), jnp.int32),
            pltpu.VMEM((NCH * C,), jnp.float32),
            pltpu.VMEM((C, F), jnp.float32),
            pltpu.VMEM_SHARED((N_PAD, F), jnp.float32),
            pltpu.SemaphoreType.DMA,
        ],
    )(body)


_scatter32 = _make_scatter_kernel(32)
_scatter16 = _make_scatter_kernel(16)


_EB = 12800


def _pae_body(x_ref, w1_ref, b1_ref, g_ref, bt_ref, w2_ref, b2_ref, out_ref):
    x = x_ref[...]
    w1 = w1_ref[...]
    w2 = w2_ref[...]

    def mlp(xx):
        h = jnp.maximum(jnp.dot(xx, w1, preferred_element_type=jnp.float32)
                        + b1_ref[...], 0.0)
        h = h * g_ref[...] + bt_ref[...]
        return jnp.dot(h, w2, preferred_element_type=jnp.float32) + b2_ref[...]

    h1 = mlp(x[:, :EDGENET_DIM // 2])
    h2 = mlp(x[:, EDGENET_DIM // 2:])
    dotp = jnp.sum(h1 * h2, axis=1)
    n1 = jnp.sqrt(jnp.sum(h1 * h1, axis=1))
    n2 = jnp.sqrt(jnp.sum(h2 * h2, axis=1))
    cos = dotp / (n1 * n2 + 1e-8)
    out_ref[...] = ((cos + 1.0) * 0.5).reshape(1, _EB // 128, 128)


def _pae(edgenet_input, w1, b1, geff, bt, w2, b2):
    grid = E // _EB
    out = pl.pallas_call(
        _pae_body,
        grid=(grid,),
        in_specs=[
            pl.BlockSpec((_EB, EDGENET_DIM), lambda i: (i, 0)),
            pl.BlockSpec((EDGENET_DIM // 2, PAE_HID), lambda i: (0, 0)),
            pl.BlockSpec((PAE_HID,), lambda i: (0,)),
            pl.BlockSpec((PAE_HID,), lambda i: (0,)),
            pl.BlockSpec((PAE_HID,), lambda i: (0,)),
            pl.BlockSpec((PAE_HID, PAE_HID), lambda i: (0, 0)),
            pl.BlockSpec((PAE_HID,), lambda i: (0,)),
        ],
        out_specs=pl.BlockSpec((1, _EB // 128, 128), lambda i: (i, 0, 0)),
        out_shape=jax.ShapeDtypeStruct((E // _EB, _EB // 128, 128),
                                       jnp.float32),
    )(edgenet_input, w1, b1, geff, bt, w2, b2)
    return out.reshape(E)


def _dinv_body(deg_ref, out_ref):
    deg = deg_ref[0] + deg_ref[1]
    out_ref[...] = jnp.where(deg > 0, 1.0 / jnp.sqrt(deg), 0.0)


def _dinv(deg_p):
    out = pl.pallas_call(
        _dinv_body,
        out_shape=jax.ShapeDtypeStruct((N_PAD // 128, 128), jnp.float32),
    )(deg_p.reshape(2, N_PAD // 128, 128))
    return out.reshape(N_PAD, 1)


def _prep_body(x_ref, wp_ref, wb_ref, dinv_ref, p_ref, b_ref):
    x = x_ref[...]
    p_ref[...] = dinv_ref[...] * jnp.dot(x, wp_ref[...],
                                         preferred_element_type=jnp.float32)
    b_ref[...] = jnp.dot(x, wb_ref[...], preferred_element_type=jnp.float32)


def _prep(x, wp, wb, dinv_col):
    din = x.shape[1]
    return pl.pallas_call(
        _prep_body,
        out_shape=[jax.ShapeDtypeStruct((N_PAD, 2 * NHID), jnp.float32),
                   jax.ShapeDtypeStruct((N_PAD, NHID), jnp.float32)],
    )(x, wp, wb, dinv_col)


def _mid_body(z_ref, dinv_ref, q_ref, za_ref):
    z = z_ref[0] + z_ref[1]
    dinv = dinv_ref[...]
    za_ref[...] = -dinv * z[:, :NHID]
    q_ref[...] = -(dinv * dinv) * z[:, NHID:]


def _mid(z_p, dinv_col):
    return pl.pallas_call(
        _mid_body,
        out_shape=[jax.ShapeDtypeStruct((N_PAD, NHID), jnp.float32),
                   jax.ShapeDtypeStruct((N_PAD, NHID), jnp.float32)],
    )(z_p, dinv_col)


def _bound_body(b_ref, za_ref, s_ref, dinv_ref, t_ref, wp_ref, wb_ref,
                t_out, p_ref, bo_ref, *, first):
    dinv = dinv_ref[...]
    x = jnp.maximum(
        b_ref[...] + za_ref[...] - 2.0 * dinv * (s_ref[0] + s_ref[1]), 0.0)
    t_new = x if first else t_ref[...] + x
    t_out[...] = t_new
    p_ref[...] = dinv * jnp.dot(t_new, wp_ref[...],
                                preferred_element_type=jnp.float32)
    bo_ref[...] = jnp.dot(t_new, wb_ref[...],
                          preferred_element_type=jnp.float32)


def _boundary(b, za, s_p, dinv_col, t_prev, wp, wb, first):
    return pl.pallas_call(
        functools.partial(_bound_body, first=first),
        out_shape=[jax.ShapeDtypeStruct((N_PAD, NHID), jnp.float32),
                   jax.ShapeDtypeStruct((N_PAD, 2 * NHID), jnp.float32),
                   jax.ShapeDtypeStruct((N_PAD, NHID), jnp.float32)],
    )(b, za, s_p, dinv_col, t_prev, wp, wb)


def _final_body(b_ref, za_ref, s_ref, dinv_ref, w1_ref, b1_ref, g_ref,
                bt_ref, w2_ref, b2_ref, out_ref):
    dinv = dinv_ref[...]
    x = jnp.maximum(
        b_ref[...] + za_ref[...] - 2.0 * dinv * (s_ref[0] + s_ref[1]), 0.0)
    h = jnp.maximum(jnp.dot(x, w1_ref[...],
                            preferred_element_type=jnp.float32)
                    + b1_ref[...], 0.0)
    h = h * g_ref[...] + bt_ref[...]
    out_ref[...] = jnp.dot(h, w2_ref[...],
                           preferred_element_type=jnp.float32) + b2_ref[...]


def _final(b, za, s_p, dinv_col, w1, b1, geff, bt, w2p, b2p):
    return pl.pallas_call(
        _final_body,
        out_shape=jax.ShapeDtypeStruct((N_PAD, 8), jnp.float32),
    )(b, za, s_p, dinv_col, w1, b1, geff, bt, w2p, b2p)


def kernel(features, edge_index, edgenet_input, cheb_w0, cheb_w_rest,
           pae_w1, pae_b1, pae_g, pae_bt, pae_w2, pae_b2,
           cls_w1, cls_b1, cls_g, cls_bt, cls_w2, cls_b2):
    f32 = jnp.float32
    inv_bn = 1.0 / jnp.sqrt(1.0 + EPS)
    pae_geff = pae_g * inv_bn
    cls_geff = cls_g * inv_bn

    ew = _pae(edgenet_input, pae_w1, pae_b1, pae_geff, pae_bt, pae_w2, pae_b2)

    npad = E_PAD - E
    pad_idx = (jnp.arange(npad, dtype=jnp.int32) % N)
    src_p = jnp.concatenate([edge_index[0], pad_idx]).reshape(NW, NCH, C)
    dst_p = jnp.concatenate([edge_index[1], pad_idx]).reshape(NW, NCH, C)
    ew_p = jnp.concatenate([ew, jnp.zeros((npad,), f32)]).reshape(NW, NCH, C)
    ew_flat = ew_p.reshape(NW, NCH * C)

    zeros1 = jnp.zeros((N_PAD,), f32)
    zeros16 = jnp.zeros((N_PAD, NHID), f32)
    zeros32 = jnp.zeros((N_PAD, 2 * NHID), f32)

    deg_p = _deg_kernel(src_p, ew_p, zeros1)
    dinv_col = _dinv(deg_p)

    feat_p = jnp.pad(features, ((0, N_PAD - N), (0, 0)))
    wps = [jnp.concatenate([cheb_w0[1], cheb_w0[2]], axis=1)]
    wbs = [cheb_w0[0] - cheb_w0[2]]
    for i in range(NGL - 1):
        wps.append(jnp.concatenate([cheb_w_rest[i, 1], cheb_w_rest[i, 2]],
                                   axis=1))
        wbs.append(cheb_w_rest[i, 0] - cheb_w_rest[i, 2])

    def _A_jnp(u, F):
        m = ew[:, None] * u[edge_index[0]]
        seg = jax.ops.segment_sum(m, edge_index[1], num_segments=N_PAD)
        return jnp.stack([seg, jnp.zeros((N_PAD, F), jnp.float32)])

    p_tab, b_cur = _prep(feat_p, wps[0], wbs[0], dinv_col)
    t_prev = zeros16
    for l in range(NGL):
        z_p = _A_jnp(p_tab, 2 * NHID)
        q_tab, za = _mid(z_p, dinv_col)
        s_p = _A_jnp(q_tab, NHID)
        if l < NGL - 1:
            t_prev, p_tab, b_cur = _boundary(
                b_cur, za, s_p, dinv_col, t_prev, wps[l + 1], wbs[l + 1],
                first=(l == 0))
        else:
            w2p = jnp.pad(cls_w2, ((0, 0), (0, 8 - cls_w2.shape[1])))
            b2p = jnp.pad(cls_b2, (0, 8 - cls_b2.shape[0]))
            logits8 = _final(b_cur, za, s_p, dinv_col, cls_w1, cls_b1,
                             cls_geff, cls_bt, w2p, b2p)
    logits = logits8[:N, :2]
    return (logits, ew)


# speedup vs baseline: 7.5696x; 5.5513x over previous
import functools

import jax
import jax.numpy as jnp
from jax import lax
from jax.experimental import pallas as pl
from jax.experimental.pallas import tpu as pltpu
from jax.experimental.pallas import tpu_sc as plsc

N = 10000
E = 320000
D_IN = 128
NHID = 16
NGL = 4
EDGENET_DIM = 32
PAE_HID = 32
EPS = 1e-5

NW = 32
C = 128
NCH = 79
E_PAD = NW * NCH * C
N_PAD = 10240
SLICE = N_PAD // 16

_SC_MESH = dict(core_axis_name="c", subcore_axis_name="s")


def _deg_body(src_hbm, ew_hbm, zero_hbm, out_hbm, src_v, ew_v, acc_sh):
    cid = lax.axis_index("c")
    sid = lax.axis_index("s")
    wid = sid * 2 + cid
    pltpu.sync_copy(src_hbm.at[wid], src_v)
    pltpu.sync_copy(ew_hbm.at[wid], ew_v)
    pltpu.sync_copy(zero_hbm.at[pl.ds(sid * SLICE, SLICE)],
                    acc_sh.at[pl.ds(sid * SLICE, SLICE)])
    plsc.subcore_barrier()

    def chunk(ch, carry):
        pltpu.sync_copy(ew_v.at[ch], acc_sh.at[src_v.at[ch]], add=True)
        return carry

    lax.fori_loop(0, NCH, chunk, 0)
    plsc.subcore_barrier()
    pltpu.sync_copy(acc_sh.at[pl.ds(sid * SLICE, SLICE)],
                    out_hbm.at[cid, pl.ds(sid * SLICE, SLICE)])


_deg_kernel = functools.partial(
    pl.kernel,
    mesh=plsc.VectorSubcoreMesh(**_SC_MESH),
    out_type=jax.ShapeDtypeStruct((2, N_PAD), jnp.float32),
    scratch_types=[
        pltpu.VMEM((NCH, C), jnp.int32),
        pltpu.VMEM((NCH, C), jnp.float32),
        pltpu.VMEM_SHARED((N_PAD,), jnp.float32),
    ],
)(_deg_body)


def _make_scatter_kernel(F):
    def body(v_hbm, src_hbm, dst_hbm, ew_hbm, zero_hbm, out_hbm,
             src_v, dst_v, ew_v, rows_v, acc_sh, sem):
        cid = lax.axis_index("c")
        sid = lax.axis_index("s")
        wid = sid * 2 + cid
        pltpu.sync_copy(src_hbm.at[wid], src_v)
        pltpu.sync_copy(dst_hbm.at[wid], dst_v)
        pltpu.sync_copy(ew_hbm.at[wid], ew_v)
        pltpu.sync_copy(zero_hbm.at[pl.ds(sid * SLICE, SLICE)],
                        acc_sh.at[pl.ds(sid * SLICE, SLICE)])
        plsc.subcore_barrier()

        gdn = lax.GatherDimensionNumbers(
            offset_dims=(), collapsed_slice_dims=(0,), start_index_map=(0,))

        def splat(vec, j):
            idx = jnp.full((16, 1), j, jnp.int32)
            return lax.gather(vec, idx, gdn, (1,),
                              mode=lax.GatherScatterMode.PROMISE_IN_BOUNDS)

        def chunk(ch, carry):
            pltpu.async_copy(v_hbm.at[src_v.at[ch]], rows_v, sem).wait()

            def group(g, c2):
                ew16 = ew_v[pl.ds(ch * C + g * 16, 16)]
                for j in range(16):
                    nb = splat(ew16, j)
                    e = g * 16 + j
                    for h in range(F // 16):
                        sl = pl.ds(h * 16, 16)
                        rows_v[e, sl] = rows_v[e, sl] * nb
                return c2

            lax.fori_loop(0, C // 16, group, 0)
            pltpu.sync_copy(rows_v, acc_sh.at[dst_v.at[ch]], add=True)
            return carry

        lax.fori_loop(0, NCH, chunk, 0)
        plsc.subcore_barrier()
        pltpu.sync_copy(acc_sh.at[pl.ds(sid * SLICE, SLICE)],
                        out_hbm.at[cid, pl.ds(sid * SLICE, SLICE)])

    return functools.partial(
        pl.kernel,
        mesh=plsc.VectorSubcoreMesh(**_SC_MESH),
        out_type=jax.ShapeDtypeStruct((2, N_PAD, 128), jnp.float32),
        scratch_types=[
            pltpu.VMEM((NCH, C), jnp.int32),
            pltpu.VMEM((NCH, C), jnp.int32),
            pltpu.VMEM((NCH * C,), jnp.float32),
            pltpu.VMEM((C, 128), jnp.float32),
            pltpu.VMEM_SHARED((N_PAD, 128), jnp.float32),
            pltpu.SemaphoreType.DMA,
        ],
    )(body)


_scatter128 = _make_scatter_kernel(128)
_scatter16 = _make_scatter_kernel(16)


_EB = 12800


def _pae_body(x_ref, w1_ref, b1_ref, g_ref, bt_ref, w2_ref, b2_ref, out_ref):
    x = x_ref[...]
    w1 = w1_ref[...]
    w2 = w2_ref[...]

    def mlp(xx):
        h = jnp.maximum(jnp.dot(xx, w1, preferred_element_type=jnp.float32)
                        + b1_ref[...], 0.0)
        h = h * g_ref[...] + bt_ref[...]
        return jnp.dot(h, w2, preferred_element_type=jnp.float32) + b2_ref[...]

    h1 = mlp(x[:, :EDGENET_DIM // 2])
    h2 = mlp(x[:, EDGENET_DIM // 2:])
    dotp = jnp.sum(h1 * h2, axis=1)
    n1 = jnp.sqrt(jnp.sum(h1 * h1, axis=1))
    n2 = jnp.sqrt(jnp.sum(h2 * h2, axis=1))
    cos = dotp / (n1 * n2 + 1e-8)
    out_ref[...] = ((cos + 1.0) * 0.5).reshape(1, _EB // 128, 128)


def _pae(edgenet_input, w1, b1, geff, bt, w2, b2):
    grid = E // _EB
    out = pl.pallas_call(
        _pae_body,
        grid=(grid,),
        in_specs=[
            pl.BlockSpec((_EB, EDGENET_DIM), lambda i: (i, 0)),
            pl.BlockSpec((EDGENET_DIM // 2, PAE_HID), lambda i: (0, 0)),
            pl.BlockSpec((PAE_HID,), lambda i: (0,)),
            pl.BlockSpec((PAE_HID,), lambda i: (0,)),
            pl.BlockSpec((PAE_HID,), lambda i: (0,)),
            pl.BlockSpec((PAE_HID, PAE_HID), lambda i: (0, 0)),
            pl.BlockSpec((PAE_HID,), lambda i: (0,)),
        ],
        out_specs=pl.BlockSpec((1, _EB // 128, 128), lambda i: (i, 0, 0)),
        out_shape=jax.ShapeDtypeStruct((E // _EB, _EB // 128, 128),
                                       jnp.float32),
    )(edgenet_input, w1, b1, geff, bt, w2, b2)
    return out.reshape(E)


def _dinv_body(deg_ref, out_ref):
    deg = deg_ref[0] + deg_ref[1]
    out_ref[...] = jnp.where(deg > 0, 1.0 / jnp.sqrt(deg), 0.0)


def _dinv(deg_p):
    out = pl.pallas_call(
        _dinv_body,
        out_shape=jax.ShapeDtypeStruct((N_PAD // 128, 128), jnp.float32),
    )(deg_p.reshape(2, N_PAD // 128, 128))
    return out.reshape(N_PAD, 1)


def _prep_body(x_ref, dinv_ref, p_ref):
    p_ref[...] = dinv_ref[...] * x_ref[...]


def _prep(x, dinv_col):
    return pl.pallas_call(
        _prep_body,
        out_shape=jax.ShapeDtypeStruct((N_PAD, 128), jnp.float32),
    )(x, dinv_col)


def _make_mid(w):
    def body(z_ref, dinv_ref, u1_ref, q_ref):
        dinv = dinv_ref[...]
        u1 = -dinv * (z_ref[0, :, :w] + z_ref[1, :, :w])
        u1_ref[...] = u1
        q = dinv * u1
        if w < 128:
            q = jnp.pad(q, ((0, 0), (0, 128 - w)))
        q_ref[...] = q

    def call(z_p, dinv_col):
        return pl.pallas_call(
            body,
            out_shape=[jax.ShapeDtypeStruct((N_PAD, w), jnp.float32),
                       jax.ShapeDtypeStruct((N_PAD, 128), jnp.float32)],
        )(z_p, dinv_col)
    return call


_mid128 = _make_mid(128)
_mid16 = _make_mid(NHID)


def _make_bound(w, first, last):
    def body(s_ref, u1_ref, t_ref, tp_ref, dinv_ref, w0_ref, w1_ref, w2_ref,
             x_ref, tn_ref, q_ref):
        dinv = dinv_ref[...]
        u2 = -dinv * (s_ref[0, :, :w] + s_ref[1, :, :w])
        t = t_ref[...]
        out = (jnp.dot(t, w0_ref[...], preferred_element_type=jnp.float32)
               + jnp.dot(u1_ref[...], w1_ref[...],
                         preferred_element_type=jnp.float32)
               + jnp.dot(2.0 * u2 - t, w2_ref[...],
                         preferred_element_type=jnp.float32))
        x = jnp.maximum(out, 0.0)
        x_ref[...] = x
        if not last:
            t_new = x if first else tp_ref[...] + x
            tn_ref[...] = t_new
            q_ref[...] = jnp.pad(dinv * t_new, ((0, 0), (0, 128 - NHID)))

    def call(s_p, u1, t, t_prev, dinv_col, w0, w1, w2):
        return pl.pallas_call(
            body,
            out_shape=[jax.ShapeDtypeStruct((N_PAD, NHID), jnp.float32),
                       jax.ShapeDtypeStruct((N_PAD, NHID), jnp.float32),
                       jax.ShapeDtypeStruct((N_PAD, 128), jnp.float32)],
        )(s_p, u1, t, t_prev, dinv_col, w0, w1, w2)
    return call


_bound1 = _make_bound(128, True, False)
_bound2 = _make_bound(NHID, False, False)
_bound_last = _make_bound(NHID, False, True)


def _final_body(x_ref, w1_ref, b1_ref, g_ref, bt_ref, w2_ref, b2_ref,
                out_ref):
    h = jnp.maximum(jnp.dot(x_ref[...], w1_ref[...],
                            preferred_element_type=jnp.float32)
                    + b1_ref[...], 0.0)
    h = h * g_ref[...] + bt_ref[...]
    out_ref[...] = jnp.dot(h, w2_ref[...],
                           preferred_element_type=jnp.float32) + b2_ref[...]


def _final(x, w1, b1, geff, bt, w2p, b2p):
    return pl.pallas_call(
        _final_body,
        out_shape=jax.ShapeDtypeStruct((N_PAD, 8), jnp.float32),
    )(x, w1, b1, geff, bt, w2p, b2p)


def kernel(features, edge_index, edgenet_input, cheb_w0, cheb_w_rest,
           pae_w1, pae_b1, pae_g, pae_bt, pae_w2, pae_b2,
           cls_w1, cls_b1, cls_g, cls_bt, cls_w2, cls_b2):
    f32 = jnp.float32
    inv_bn = 1.0 / jnp.sqrt(1.0 + EPS)
    pae_geff = pae_g * inv_bn
    cls_geff = cls_g * inv_bn

    ew = _pae(edgenet_input, pae_w1, pae_b1, pae_geff, pae_bt, pae_w2, pae_b2)

    npad = E_PAD - E
    pad_idx = (jnp.arange(npad, dtype=jnp.int32) % N)
    src_p = jnp.concatenate([edge_index[0], pad_idx]).reshape(NW, NCH, C)
    dst_p = jnp.concatenate([edge_index[1], pad_idx]).reshape(NW, NCH, C)
    ew_p = jnp.concatenate([ew, jnp.zeros((npad,), f32)]).reshape(NW, NCH, C)
    ew_flat = ew_p.reshape(NW, NCH * C)

    zeros1 = jnp.zeros((N_PAD,), f32)
    zeros16 = jnp.zeros((N_PAD, NHID), f32)
    zeros128 = jnp.zeros((N_PAD, 128), f32)

    deg_p = _deg_kernel(src_p, ew_p, zeros1)
    dinv_col = _dinv(deg_p)

    feat_p = jnp.pad(features, ((0, N_PAD - N), (0, 0)))
    q_tab = _prep(feat_p, dinv_col)
    t_cur = feat_p
    for l in range(NGL):
        wide = (l == 0)
        scat = _scatter128 if wide else _scatter16
        mid = _mid128 if wide else _mid16
        w_l = cheb_w0 if l == 0 else cheb_w_rest[l - 1]
        z_p = scat(q_tab, src_p, dst_p, ew_flat, zeros128)
        u1, q2 = mid(z_p, dinv_col)
        s_p = scat(q2, src_p, dst_p, ew_flat, zeros128)
        if l == 0:
            bound = _bound1
        elif l < NGL - 1:
            bound = _bound2
        else:
            bound = _bound_last
        x_l, t_new, q_tab = bound(s_p, u1, t_cur, t_cur, dinv_col,
                                  w_l[0], w_l[1], w_l[2])
        t_cur = t_new
    w2p = jnp.pad(cls_w2, ((0, 0), (0, 8 - cls_w2.shape[1])))
    b2p = jnp.pad(cls_b2, (0, 8 - cls_b2.shape[0]))
    logits8 = _final(x_l, cls_w1, cls_b1, cls_geff, cls_bt, w2p, b2p)
    logits = logits8[:N, :2]
    return (logits, ew)
